# baseline (speedup 1.0000x reference)
import functools

import jax
import jax.numpy as jnp
from jax import lax
from jax.experimental import pallas as pl
from jax.experimental.pallas import tpu as pltpu
from jax.experimental.pallas import tpu_sc as plsc

B = 8
M = 32
N_PAD = 20480
ROWS = 160
NW = 32
SC_ROWS = 112
TC_ROWS = ROWS - SC_ROWS
ROW_BLK = 16
CHUNK = SC_ROWS * 128 // NW
VECS = CHUNK // 16


def _prep_body(a_ref, ann_ref, at_ref, annc_ref):
    ay1 = a_ref[0]
    ax1 = a_ref[1]
    ay2 = a_ref[2]
    ax2 = a_ref[3]
    aw = ax2 - ax1
    ah = ay2 - ay1
    at_ref[0] = ax1
    at_ref[1] = ax2
    at_ref[2] = ay1
    at_ref[3] = ay2
    at_ref[4] = jnp.log(aw)
    at_ref[5] = jnp.log(ah)

    bx1 = ann_ref[:, 0, :]
    by1 = ann_ref[:, 1, :]
    bx2 = ann_ref[:, 2, :]
    by2 = ann_ref[:, 3, :]
    bw = bx2 - bx1
    bh = by2 - by1
    annc_ref[:, 0, :] = bx1
    annc_ref[:, 1, :] = bx2
    annc_ref[:, 2, :] = by1
    annc_ref[:, 3, :] = by2
    annc_ref[:, 4, :] = bw * bh
    annc_ref[:, 5, :] = bx1 + 0.5 * bw
    annc_ref[:, 6, :] = by1 + 0.5 * bh
    annc_ref[:, 7, :] = jnp.log(jnp.maximum(bw, 1.0))
    annc_ref[:, 8, :] = jnp.log(jnp.maximum(bh, 1.0))


def _sc_body(at_hbm, exp_hbm, reg_hbm, out_hbm,
             at_v, exp_v, reg_v, outb_v, sem):
    c = lax.axis_index("c")
    s = lax.axis_index("s")
    wid = s * 2 + c
    h1 = pltpu.async_copy(at_hbm.at[wid], at_v, sem)
    h2 = pltpu.async_copy(exp_hbm, exp_v, sem)
    h3 = pltpu.async_copy(reg_hbm.at[wid], reg_v, sem)
    h1.wait()
    h2.wait()
    h3.wait()
    zero = jnp.zeros((16,), jnp.float32)
    for b0 in range(B):
        outb_v[b0, 0] = zero
        outb_v[b0, 1] = zero
    NV = 4
    NI = VECS // NV

    def vec_step(t, _):
        b = t // NI
        i = t % NI
        sls = [pl.ds(pl.multiple_of(i * (16 * NV) + 16 * v, 16), 16)
               for v in range(NV)]
        ax1 = [at_v[0, sl] for sl in sls]
        ax2 = [at_v[1, sl] for sl in sls]
        ay1 = [at_v[2, sl] for sl in sls]
        ay2 = [at_v[3, sl] for sl in sls]
        aw = [ax2[v] - ax1[v] for v in range(NV)]
        ah = [ay2[v] - ay1[v] for v in range(NV)]
        area_a = [ah[v] * aw[v] for v in range(NV)]
        best_iou = [jnp.full((16,), -1.0, jnp.float32) for _ in range(NV)]
        gcx = [zero] * NV
        gcy = [zero] * NV
        glw = [zero] * NV
        glh = [zero] * NV
        for m in range(M):
            ms = pl.ds(m * 16, 16)
            bx1 = exp_v[b, 0, ms]
            bx2 = exp_v[b, 1, ms]
            by1 = exp_v[b, 2, ms]
            by2 = exp_v[b, 3, ms]
            area_b = exp_v[b, 4, ms]
            e5 = exp_v[b, 5, ms]
            e6 = exp_v[b, 6, ms]
            e7 = exp_v[b, 7, ms]
            e8 = exp_v[b, 8, ms]
            for v in range(NV):
                iw = jnp.maximum(
                    jnp.minimum(ax2[v], bx2) - jnp.maximum(ax1[v], bx1), 0.0)
                ih = jnp.maximum(
                    jnp.minimum(ay2[v], by2) - jnp.maximum(ay1[v], by1), 0.0)
                inter = iw * ih
                ua = area_a[v] + area_b - inter
                iou = inter / ua
                cond = iou > best_iou[v]
                best_iou[v] = jnp.where(cond, iou, best_iou[v])
                gcx[v] = jnp.where(cond, e5, gcx[v])
                gcy[v] = jnp.where(cond, e6, gcy[v])
                glw[v] = jnp.where(cond, e7, glw[v])
                glh[v] = jnp.where(cond, e8, glh[v])
        acc_l = zero
        acc_c = zero
        for v in range(NV):
            sl = sls[v]
            acx = ax1[v] + 0.5 * aw[v]
            acy = ay1[v] + 0.5 * ah[v]
            law = at_v[4, sl]
            lah = at_v[5, sl]
            positive = best_iou[v] >= 0.5
            dy = (gcy[v] - acy) / ah[v]
            dx = (gcx[v] - acx) / aw[v]
            dh = glh[v] - lah
            dw = glw[v] - law
            loss = zero
            for j, t2 in enumerate((dy, dx, dh, dw)):
                diff = jnp.abs(t2 - reg_v[b, j, sl])
                loss = loss + jnp.where(
                    diff <= 1.0 / 9.0, 4.5 * diff * diff, diff - 0.5 / 9.0)
            acc_l = acc_l + jnp.where(positive, loss, 0.0)
            acc_c = acc_c + jnp.where(positive, 1.0, 0.0)
        plsc.addupdate(outb_v.at[b, 0], acc_l)
        plsc.addupdate(outb_v.at[b, 1], acc_c)
        return 0

    lax.fori_loop(0, B * NI, vec_step, 0)
    pltpu.sync_copy(outb_v, out_hbm.at[wid])


def _tc_main_body(a_ref, lg_ref, reg_ref, ann_ref, out_ref):
    b = pl.program_id(0)
    ay1 = a_ref[0]
    ax1 = a_ref[1]
    ay2 = a_ref[2]
    ax2 = a_ref[3]
    aw = ax2 - ax1
    ah = ay2 - ay1
    acx = ax1 + 0.5 * aw
    acy = ay1 + 0.5 * ah
    area_a = ah * aw
    law = lg_ref[0]
    lah = lg_ref[1]

    zeros = jnp.zeros_like(aw)

    def m_step(m, carry):
        best_iou, scx, scy, slw, slh = carry
        bx1 = ann_ref[b, m, 0]
        by1 = ann_ref[b, m, 1]
        bx2 = ann_ref[b, m, 2]
        by2 = ann_ref[b, m, 3]
        bw = bx2 - bx1
        bh = by2 - by1
        area_b = bw * bh
        gcx = bx1 + 0.5 * bw
        gcy = by1 + 0.5 * bh
        glw = jnp.log(jnp.maximum(bw, 1.0))
        glh = jnp.log(jnp.maximum(bh, 1.0))
        iw = jnp.maximum(jnp.minimum(ax2, bx2) - jnp.maximum(ax1, bx1), 0.0)
        ih = jnp.maximum(jnp.minimum(ay2, by2) - jnp.maximum(ay1, by1), 0.0)
        inter = iw * ih
        iou = inter / (area_a + area_b - inter)
        cond = iou > best_iou
        best_iou = jnp.where(cond, iou, best_iou)
        scx = jnp.where(cond, gcx, scx)
        scy = jnp.where(cond, gcy, scy)
        slw = jnp.where(cond, glw, slw)
        slh = jnp.where(cond, glh, slh)
        return best_iou, scx, scy, slw, slh

    init = (zeros - 1.0, zeros, zeros, zeros, zeros)
    best_iou, scx, scy, slw, slh = lax.fori_loop(0, M, m_step, init)

    positive = best_iou >= 0.5
    dy = (scy - acy) / ah
    dx = (scx - acx) / aw
    dh = slh - lah
    dw = slw - law
    loss = zeros
    for j, t in enumerate((dy, dx, dh, dw)):
        diff = jnp.abs(t - reg_ref[0, j])
        loss += jnp.where(diff <= 1.0 / 9.0, 4.5 * diff * diff, diff - 0.5 / 9.0)
    loss = jnp.where(positive, loss, 0.0)

    @pl.when(pl.program_id(1) == 0)
    def _():
        out_ref[b, 0] = 0.0
        out_ref[b, 1] = 0.0

    out_ref[b, 0] += jnp.sum(loss)
    out_ref[b, 1] += jnp.sum(positive.astype(jnp.float32))


def _combine_body(p_ref, q_ref, out_ref):
    total = 0.0
    for b in range(B):
        ls = jnp.sum(p_ref[:, b, 0, :]) + q_ref[b, 0]
        np_ = jnp.sum(p_ref[:, b, 1, :]) + q_ref[b, 1]
        total += jnp.where(np_ > 0.0, ls / (4.0 * jnp.maximum(np_, 1.0)), 0.0)
    out_ref[0, 0] = total * (50.0 / B)


@jax.jit
def kernel(regressions, anchors, annotations):
    n = anchors.shape[1]
    pad_box = jnp.array([-1e4, -1e4, -1e4 + 1.0, -1e4 + 1.0], jnp.float32)
    a = jnp.concatenate(
        [anchors[0], jnp.broadcast_to(pad_box, (N_PAD - n, 4))], axis=0
    )
    a_t = a.T.reshape(4, ROWS, 128)
    ann_t = annotations.transpose(0, 2, 1)
    reg = jnp.concatenate(
        [regressions, jnp.zeros((B, N_PAD - n, 4), jnp.float32)], axis=1
    )
    reg_t = reg.transpose(0, 2, 1)

    at3, annc = pl.pallas_call(
        _prep_body,
        out_shape=[
            jax.ShapeDtypeStruct((6, ROWS, 128), jnp.float32),
            jax.ShapeDtypeStruct((B, 9, M), jnp.float32),
        ],
    )(a_t, ann_t)
    at = (at3[:, :SC_ROWS, :].reshape(6, NW, CHUNK)
          .transpose(1, 0, 2))
    ann_exp = jnp.broadcast_to(
        annc[:, :, :, None], (B, 9, M, 16)).reshape(B, 9, M * 16)
    reg_w = (reg_t[:, :, :SC_ROWS * 128].reshape(B, 4, NW, CHUNK)
             .transpose(2, 0, 1, 3))
    reg4 = reg_t.reshape(B, 4, ROWS, 128)

    mesh = plsc.VectorSubcoreMesh(core_axis_name="c", subcore_axis_name="s")
    sc_main = functools.partial(
        pl.kernel,
        mesh=mesh,
        out_type=jax.ShapeDtypeStruct((NW, B, 2, 16), jnp.float32),
        scratch_types=[
            pltpu.VMEM((6, CHUNK), jnp.float32),
            pltpu.VMEM((B, 9, M * 16), jnp.float32),
            pltpu.VMEM((B, 4, CHUNK), jnp.float32),
            pltpu.VMEM((B, 2, 16), jnp.float32),
            pltpu.SemaphoreType.DMA,
        ],
    )(_sc_body)
    partials = sc_main(at, ann_exp, reg_w)

    sc_nb = SC_ROWS // ROW_BLK
    tc_partials = pl.pallas_call(
        _tc_main_body,
        grid=(B, TC_ROWS // ROW_BLK),
        in_specs=[
            pl.BlockSpec((4, ROW_BLK, 128), lambda b, i: (0, i + sc_nb, 0)),
            pl.BlockSpec((2, ROW_BLK, 128), lambda b, i: (2, i + sc_nb, 0)),
            pl.BlockSpec((1, 4, ROW_BLK, 128),
                         lambda b, i: (b, 0, i + sc_nb, 0)),
            pl.BlockSpec(memory_space=pltpu.SMEM),
        ],
        out_specs=pl.BlockSpec(memory_space=pltpu.SMEM),
        out_shape=jax.ShapeDtypeStruct((B, 2), jnp.float32),
    )(a_t, at3, reg4, annotations)

    out = pl.pallas_call(
        _combine_body,
        in_specs=[
            pl.BlockSpec(),
            pl.BlockSpec(memory_space=pltpu.SMEM),
        ],
        out_specs=pl.BlockSpec(memory_space=pltpu.SMEM),
        out_shape=jax.ShapeDtypeStruct((1, 1), jnp.float32),
    )(partials, tc_partials)
    return out.reshape(1)


# speedup vs baseline: 1.0142x; 1.0142x over previous
import functools

import jax
import jax.numpy as jnp
from jax import lax
from jax.experimental import pallas as pl
from jax.experimental.pallas import tpu as pltpu
from jax.experimental.pallas import tpu_sc as plsc

B = 8
M = 32
N_PAD = 20480
ROWS = 160
NW = 32
SC_ROWS = 112
TC_ROWS = ROWS - SC_ROWS
ROW_BLK = 16
CHUNK = SC_ROWS * 128 // NW
VECS = CHUNK // 16


def _prep_body(a_ref, ann_ref, at_ref, annc_ref):
    ay1 = a_ref[0]
    ax1 = a_ref[1]
    ay2 = a_ref[2]
    ax2 = a_ref[3]
    aw = ax2 - ax1
    ah = ay2 - ay1
    at_ref[0] = ax1
    at_ref[1] = ax2
    at_ref[2] = ay1
    at_ref[3] = ay2
    at_ref[4] = jnp.log(aw)
    at_ref[5] = jnp.log(ah)

    bx1 = ann_ref[:, 0, :]
    by1 = ann_ref[:, 1, :]
    bx2 = ann_ref[:, 2, :]
    by2 = ann_ref[:, 3, :]
    bw = bx2 - bx1
    bh = by2 - by1
    annc_ref[:, 0, :] = bx1
    annc_ref[:, 1, :] = bx2
    annc_ref[:, 2, :] = by1
    annc_ref[:, 3, :] = by2
    annc_ref[:, 4, :] = bw * bh
    annc_ref[:, 5, :] = bx1 + 0.5 * bw
    annc_ref[:, 6, :] = by1 + 0.5 * bh
    annc_ref[:, 7, :] = jnp.log(jnp.maximum(bw, 1.0))
    annc_ref[:, 8, :] = jnp.log(jnp.maximum(bh, 1.0))


def _sc_body(at_hbm, exp_hbm, reg_hbm, out_hbm,
             at_v, exp_v, reg_v, outb_v, sem):
    c = lax.axis_index("c")
    s = lax.axis_index("s")
    wid = s * 2 + c
    h1 = pltpu.async_copy(at_hbm.at[wid], at_v, sem)
    h2 = pltpu.async_copy(exp_hbm, exp_v, sem)
    h3 = pltpu.async_copy(reg_hbm.at[wid], reg_v, sem)
    h1.wait()
    h2.wait()
    h3.wait()
    zero = jnp.zeros((16,), jnp.float32)
    for b0 in range(B):
        outb_v[b0, 0] = zero
        outb_v[b0, 1] = zero
    NV = 4
    NI = VECS // NV

    def vec_step(t, _):
        b = t // NI
        i = t % NI
        sls = [pl.ds(pl.multiple_of(i * (16 * NV) + 16 * v, 16), 16)
               for v in range(NV)]
        ax1 = [at_v[0, sl] for sl in sls]
        ax2 = [at_v[1, sl] for sl in sls]
        ay1 = [at_v[2, sl] for sl in sls]
        ay2 = [at_v[3, sl] for sl in sls]
        aw = [ax2[v] - ax1[v] for v in range(NV)]
        ah = [ay2[v] - ay1[v] for v in range(NV)]
        area_a = [ah[v] * aw[v] for v in range(NV)]
        best_iou = [jnp.full((16,), -1.0, jnp.float32) for _ in range(NV)]
        gcx = [zero] * NV
        gcy = [zero] * NV
        glw = [zero] * NV
        glh = [zero] * NV
        for m in range(M):
            ms = pl.ds(m * 16, 16)
            bx1 = exp_v[b, 0, ms]
            bx2 = exp_v[b, 1, ms]
            by1 = exp_v[b, 2, ms]
            by2 = exp_v[b, 3, ms]
            area_b = exp_v[b, 4, ms]
            e5 = exp_v[b, 5, ms]
            e6 = exp_v[b, 6, ms]
            e7 = exp_v[b, 7, ms]
            e8 = exp_v[b, 8, ms]
            for v in range(NV):
                iw = jnp.maximum(
                    jnp.minimum(ax2[v], bx2) - jnp.maximum(ax1[v], bx1), 0.0)
                ih = jnp.maximum(
                    jnp.minimum(ay2[v], by2) - jnp.maximum(ay1[v], by1), 0.0)
                inter = iw * ih
                ua = area_a[v] + area_b - inter
                iou = inter / ua
                cond = iou > best_iou[v]
                best_iou[v] = jnp.where(cond, iou, best_iou[v])
                gcx[v] = jnp.where(cond, e5, gcx[v])
                gcy[v] = jnp.where(cond, e6, gcy[v])
                glw[v] = jnp.where(cond, e7, glw[v])
                glh[v] = jnp.where(cond, e8, glh[v])
        acc_l = zero
        acc_c = zero
        for v in range(NV):
            sl = sls[v]
            acx = ax1[v] + 0.5 * aw[v]
            acy = ay1[v] + 0.5 * ah[v]
            law = at_v[4, sl]
            lah = at_v[5, sl]
            positive = best_iou[v] >= 0.5
            dy = (gcy[v] - acy) / ah[v]
            dx = (gcx[v] - acx) / aw[v]
            dh = glh[v] - lah
            dw = glw[v] - law
            loss = zero
            for j, t2 in enumerate((dy, dx, dh, dw)):
                diff = jnp.abs(t2 - reg_v[b, j, sl])
                loss = loss + jnp.where(
                    diff <= 1.0 / 9.0, 4.5 * diff * diff, diff - 0.5 / 9.0)
            acc_l = acc_l + jnp.where(positive, loss, 0.0)
            acc_c = acc_c + jnp.where(positive, 1.0, 0.0)
        plsc.addupdate(outb_v.at[b, 0], acc_l)
        plsc.addupdate(outb_v.at[b, 1], acc_c)
        return 0

    lax.fori_loop(0, B * NI, vec_step, 0)
    pltpu.sync_copy(outb_v, out_hbm.at[wid])


def _tc_main_body(a_ref, reg_ref, ann_ref, out_ref):
    b = pl.program_id(0)
    ay1 = a_ref[0]
    ax1 = a_ref[1]
    ay2 = a_ref[2]
    ax2 = a_ref[3]
    aw = ax2 - ax1
    ah = ay2 - ay1
    acx = ax1 + 0.5 * aw
    acy = ay1 + 0.5 * ah
    area_a = ah * aw
    law = jnp.log(aw)
    lah = jnp.log(ah)

    zeros = jnp.zeros_like(aw)

    def m_step(m, carry):
        best_iou, scx, scy, slw, slh = carry
        bx1 = ann_ref[b, m, 0]
        by1 = ann_ref[b, m, 1]
        bx2 = ann_ref[b, m, 2]
        by2 = ann_ref[b, m, 3]
        bw = bx2 - bx1
        bh = by2 - by1
        area_b = bw * bh
        gcx = bx1 + 0.5 * bw
        gcy = by1 + 0.5 * bh
        glw = jnp.log(jnp.maximum(bw, 1.0))
        glh = jnp.log(jnp.maximum(bh, 1.0))
        iw = jnp.maximum(jnp.minimum(ax2, bx2) - jnp.maximum(ax1, bx1), 0.0)
        ih = jnp.maximum(jnp.minimum(ay2, by2) - jnp.maximum(ay1, by1), 0.0)
        inter = iw * ih
        iou = inter / (area_a + area_b - inter)
        cond = iou > best_iou
        best_iou = jnp.where(cond, iou, best_iou)
        scx = jnp.where(cond, gcx, scx)
        scy = jnp.where(cond, gcy, scy)
        slw = jnp.where(cond, glw, slw)
        slh = jnp.where(cond, glh, slh)
        return best_iou, scx, scy, slw, slh

    init = (zeros - 1.0, zeros, zeros, zeros, zeros)
    best_iou, scx, scy, slw, slh = lax.fori_loop(0, M, m_step, init)

    positive = best_iou >= 0.5
    dy = (scy - acy) / ah
    dx = (scx - acx) / aw
    dh = slh - lah
    dw = slw - law
    loss = zeros
    for j, t in enumerate((dy, dx, dh, dw)):
        diff = jnp.abs(t - reg_ref[0, j])
        loss += jnp.where(diff <= 1.0 / 9.0, 4.5 * diff * diff, diff - 0.5 / 9.0)
    loss = jnp.where(positive, loss, 0.0)

    @pl.when(pl.program_id(1) == 0)
    def _():
        out_ref[b, 0] = 0.0
        out_ref[b, 1] = 0.0

    out_ref[b, 0] += jnp.sum(loss)
    out_ref[b, 1] += jnp.sum(positive.astype(jnp.float32))


def _combine_body(p_ref, q_ref, out_ref):
    total = 0.0
    for b in range(B):
        ls = jnp.sum(p_ref[:, b, 0, :]) + q_ref[b, 0]
        np_ = jnp.sum(p_ref[:, b, 1, :]) + q_ref[b, 1]
        total += jnp.where(np_ > 0.0, ls / (4.0 * jnp.maximum(np_, 1.0)), 0.0)
    out_ref[0, 0] = total * (50.0 / B)


@jax.jit
def kernel(regressions, anchors, annotations):
    n = anchors.shape[1]
    pad_box = jnp.array([-1e4, -1e4, -1e4 + 1.0, -1e4 + 1.0], jnp.float32)
    a = jnp.concatenate(
        [anchors[0], jnp.broadcast_to(pad_box, (N_PAD - n, 4))], axis=0
    )
    a_t = a.T.reshape(4, ROWS, 128)
    ann_t = annotations.transpose(0, 2, 1)
    reg = jnp.concatenate(
        [regressions, jnp.zeros((B, N_PAD - n, 4), jnp.float32)], axis=1
    )
    reg_t = reg.transpose(0, 2, 1)

    at3, annc = pl.pallas_call(
        _prep_body,
        out_shape=[
            jax.ShapeDtypeStruct((6, ROWS, 128), jnp.float32),
            jax.ShapeDtypeStruct((B, 9, M), jnp.float32),
        ],
    )(a_t, ann_t)
    at = (at3[:, :SC_ROWS, :].reshape(6, NW, CHUNK)
          .transpose(1, 0, 2))
    ann_exp = jnp.broadcast_to(
        annc[:, :, :, None], (B, 9, M, 16)).reshape(B, 9, M * 16)
    reg_w = (reg_t[:, :, :SC_ROWS * 128].reshape(B, 4, NW, CHUNK)
             .transpose(2, 0, 1, 3))
    reg4 = reg_t.reshape(B, 4, ROWS, 128)

    mesh = plsc.VectorSubcoreMesh(core_axis_name="c", subcore_axis_name="s")
    sc_main = functools.partial(
        pl.kernel,
        mesh=mesh,
        out_type=jax.ShapeDtypeStruct((NW, B, 2, 16), jnp.float32),
        scratch_types=[
            pltpu.VMEM((6, CHUNK), jnp.float32),
            pltpu.VMEM((B, 9, M * 16), jnp.float32),
            pltpu.VMEM((B, 4, CHUNK), jnp.float32),
            pltpu.VMEM((B, 2, 16), jnp.float32),
            pltpu.SemaphoreType.DMA,
        ],
    )(_sc_body)
    partials = sc_main(at, ann_exp, reg_w)

    sc_nb = SC_ROWS // ROW_BLK
    tc_partials = pl.pallas_call(
        _tc_main_body,
        grid=(B, TC_ROWS // ROW_BLK),
        in_specs=[
            pl.BlockSpec((4, ROW_BLK, 128), lambda b, i: (0, i + sc_nb, 0)),
            pl.BlockSpec((1, 4, ROW_BLK, 128),
                         lambda b, i: (b, 0, i + sc_nb, 0)),
            pl.BlockSpec(memory_space=pltpu.SMEM),
        ],
        out_specs=pl.BlockSpec(memory_space=pltpu.SMEM),
        out_shape=jax.ShapeDtypeStruct((B, 2), jnp.float32),
    )(a_t, reg4, annotations)

    out = pl.pallas_call(
        _combine_body,
        in_specs=[
            pl.BlockSpec(),
            pl.BlockSpec(memory_space=pltpu.SMEM),
        ],
        out_specs=pl.BlockSpec(memory_space=pltpu.SMEM),
        out_shape=jax.ShapeDtypeStruct((1, 1), jnp.float32),
    )(partials, tc_partials)
    return out.reshape(1)


# speedup vs baseline: 1.0214x; 1.0071x over previous
import functools

import jax
import jax.numpy as jnp
from jax import lax
from jax.experimental import pallas as pl
from jax.experimental.pallas import tpu as pltpu
from jax.experimental.pallas import tpu_sc as plsc

B = 8
M = 32
N_PAD = 20480
ROWS = 160
NW = 32
CHUNK = N_PAD // NW
VECS = CHUNK // 16


def _prep_body(a_ref, ann_ref, at_ref, annc_ref):
    ay1 = a_ref[0]
    ax1 = a_ref[1]
    ay2 = a_ref[2]
    ax2 = a_ref[3]
    aw = ax2 - ax1
    ah = ay2 - ay1
    at_ref[0] = ax1
    at_ref[1] = ax2
    at_ref[2] = ay1
    at_ref[3] = ay2
    at_ref[4] = jnp.log(aw)
    at_ref[5] = jnp.log(ah)

    bx1 = ann_ref[:, 0, :]
    by1 = ann_ref[:, 1, :]
    bx2 = ann_ref[:, 2, :]
    by2 = ann_ref[:, 3, :]
    bw = bx2 - bx1
    bh = by2 - by1
    annc_ref[:, 0, :] = bx1
    annc_ref[:, 1, :] = bx2
    annc_ref[:, 2, :] = by1
    annc_ref[:, 3, :] = by2
    annc_ref[:, 4, :] = bw * bh
    annc_ref[:, 5, :] = bx1 + 0.5 * bw
    annc_ref[:, 6, :] = by1 + 0.5 * bh
    annc_ref[:, 7, :] = jnp.log(jnp.maximum(bw, 1.0))
    annc_ref[:, 8, :] = jnp.log(jnp.maximum(bh, 1.0))


def _sc_body(at_hbm, exp_hbm, reg_hbm, out_hbm,
             at_v, exp_v, reg_v, outb_v, sem):
    c = lax.axis_index("c")
    s = lax.axis_index("s")
    wid = s * 2 + c
    h1 = pltpu.async_copy(at_hbm.at[wid], at_v, sem)
    h2 = pltpu.async_copy(exp_hbm, exp_v, sem)
    h3 = pltpu.async_copy(reg_hbm.at[wid], reg_v, sem)
    h1.wait()
    h2.wait()
    h3.wait()
    zero = jnp.zeros((16,), jnp.float32)
    for b0 in range(B):
        outb_v[b0, 0] = zero
        outb_v[b0, 1] = zero
    NV = 4
    NI = VECS // NV

    def vec_step(t, _):
        b = t // NI
        i = t % NI
        sls = [pl.ds(pl.multiple_of(i * (16 * NV) + 16 * v, 16), 16)
               for v in range(NV)]
        ax1 = [at_v[0, sl] for sl in sls]
        ax2 = [at_v[1, sl] for sl in sls]
        ay1 = [at_v[2, sl] for sl in sls]
        ay2 = [at_v[3, sl] for sl in sls]
        aw = [ax2[v] - ax1[v] for v in range(NV)]
        ah = [ay2[v] - ay1[v] for v in range(NV)]
        area_a = [ah[v] * aw[v] for v in range(NV)]
        best_iou = [jnp.full((16,), -1.0, jnp.float32) for _ in range(NV)]
        gcx = [zero] * NV
        gcy = [zero] * NV
        glw = [zero] * NV
        glh = [zero] * NV
        for m in range(M):
            ms = pl.ds(m * 16, 16)
            bx1 = exp_v[b, 0, ms]
            bx2 = exp_v[b, 1, ms]
            by1 = exp_v[b, 2, ms]
            by2 = exp_v[b, 3, ms]
            area_b = exp_v[b, 4, ms]
            e5 = exp_v[b, 5, ms]
            e6 = exp_v[b, 6, ms]
            e7 = exp_v[b, 7, ms]
            e8 = exp_v[b, 8, ms]
            for v in range(NV):
                iw = jnp.maximum(
                    jnp.minimum(ax2[v], bx2) - jnp.maximum(ax1[v], bx1), 0.0)
                ih = jnp.maximum(
                    jnp.minimum(ay2[v], by2) - jnp.maximum(ay1[v], by1), 0.0)
                inter = iw * ih
                ua = area_a[v] + area_b - inter
                iou = inter / ua
                cond = iou > best_iou[v]
                best_iou[v] = jnp.where(cond, iou, best_iou[v])
                gcx[v] = jnp.where(cond, e5, gcx[v])
                gcy[v] = jnp.where(cond, e6, gcy[v])
                glw[v] = jnp.where(cond, e7, glw[v])
                glh[v] = jnp.where(cond, e8, glh[v])
        acc_l = zero
        acc_c = zero
        for v in range(NV):
            sl = sls[v]
            acx = ax1[v] + 0.5 * aw[v]
            acy = ay1[v] + 0.5 * ah[v]
            law = at_v[4, sl]
            lah = at_v[5, sl]
            positive = best_iou[v] >= 0.5
            dy = (gcy[v] - acy) / ah[v]
            dx = (gcx[v] - acx) / aw[v]
            dh = glh[v] - lah
            dw = glw[v] - law
            loss = zero
            for j, t2 in enumerate((dy, dx, dh, dw)):
                diff = jnp.abs(t2 - reg_v[b, j, sl])
                loss = loss + jnp.where(
                    diff <= 1.0 / 9.0, 4.5 * diff * diff, diff - 0.5 / 9.0)
            acc_l = acc_l + jnp.where(positive, loss, 0.0)
            acc_c = acc_c + jnp.where(positive, 1.0, 0.0)
        plsc.addupdate(outb_v.at[b, 0], acc_l)
        plsc.addupdate(outb_v.at[b, 1], acc_c)
        return 0

    lax.fori_loop(0, B * NI, vec_step, 0)
    pltpu.sync_copy(outb_v, out_hbm.at[wid])


def _combine_body(p_ref, out_ref):
    total = 0.0
    for b in range(B):
        ls = jnp.sum(p_ref[:, b, 0, :])
        np_ = jnp.sum(p_ref[:, b, 1, :])
        total += jnp.where(np_ > 0.0, ls / (4.0 * jnp.maximum(np_, 1.0)), 0.0)
    out_ref[0, 0] = total * (50.0 / B)


@jax.jit
def kernel(regressions, anchors, annotations):
    n = anchors.shape[1]
    pad_box = jnp.array([-1e4, -1e4, -1e4 + 1.0, -1e4 + 1.0], jnp.float32)
    a = jnp.concatenate(
        [anchors[0], jnp.broadcast_to(pad_box, (N_PAD - n, 4))], axis=0
    )
    a_t = a.T.reshape(4, ROWS, 128)
    ann_t = annotations.transpose(0, 2, 1)
    reg = jnp.concatenate(
        [regressions, jnp.zeros((B, N_PAD - n, 4), jnp.float32)], axis=1
    )
    reg_t = reg.transpose(0, 2, 1)

    at3, annc = pl.pallas_call(
        _prep_body,
        out_shape=[
            jax.ShapeDtypeStruct((6, ROWS, 128), jnp.float32),
            jax.ShapeDtypeStruct((B, 9, M), jnp.float32),
        ],
    )(a_t, ann_t)
    at = at3.reshape(6, NW, CHUNK).transpose(1, 0, 2)
    ann_exp = jnp.broadcast_to(
        annc[:, :, :, None], (B, 9, M, 16)).reshape(B, 9, M * 16)
    reg_w = reg_t.reshape(B, 4, NW, CHUNK).transpose(2, 0, 1, 3)

    mesh = plsc.VectorSubcoreMesh(core_axis_name="c", subcore_axis_name="s")
    sc_main = functools.partial(
        pl.kernel,
        mesh=mesh,
        out_type=jax.ShapeDtypeStruct((NW, B, 2, 16), jnp.float32),
        scratch_types=[
            pltpu.VMEM((6, CHUNK), jnp.float32),
            pltpu.VMEM((B, 9, M * 16), jnp.float32),
            pltpu.VMEM((B, 4, CHUNK), jnp.float32),
            pltpu.VMEM((B, 2, 16), jnp.float32),
            pltpu.SemaphoreType.DMA,
        ],
    )(_sc_body)
    partials = sc_main(at, ann_exp, reg_w)

    out = pl.pallas_call(
        _combine_body,
        out_specs=pl.BlockSpec(memory_space=pltpu.SMEM),
        out_shape=jax.ShapeDtypeStruct((1, 1), jnp.float32),
    )(partials)
    return out.reshape(1)
